# baseline (speedup 1.0000x reference)
import functools

import jax
import jax.numpy as jnp
import numpy as np
from jax import lax
from jax.experimental import pallas as pl
from jax.experimental.pallas import tpu as pltpu
from jax.experimental.pallas import tpu_sc as plsc

N = 2048
D = 256
H = 8
DK = 32
DFF = 1024
VOCAB = 8192
E = 65536
SQRT_D = float(np.sqrt(D))
INV_SQRT_DK = float(1.0 / np.sqrt(DK))

_INTERP = False

_NC = 2
_NS = 16
_EPT = E // _NS
_CHUNK = 512
_CW = _CHUNK * N
_SLICE = _CW // _NS
_ZW = 8192
_RPT = N // (_NC * _NS)


def _dot16(a, b):
    return jax.lax.dot_general(
        a.astype(jnp.bfloat16), b.astype(jnp.bfloat16),
        (((1,), (0,)), ((), ())), preferred_element_type=jnp.float32)


def _ln(x):
    m = jnp.mean(x, axis=-1, keepdims=True)
    v = jnp.mean((x - m) ** 2, axis=-1, keepdims=True)
    return (x - m) * jax.lax.rsqrt(v + 1e-5)


def _cbuild_body(ees, eed, cee,
                 dstv, srcv, idxv, valv, zbuf, csh, zsem, ssem):
    cid = lax.axis_index("c")
    sid = lax.axis_index("s")
    zero16 = jnp.zeros((16,), jnp.float32)

    def zb(i, c):
        zbuf[pl.ds(i * 16, 16)] = zero16
        return c
    lax.fori_loop(0, _ZW // 16, zb, 0)

    for s_hbm, d_hbm, out_hbm in ((ees, eed, cee),):
        pltpu.sync_copy(d_hbm.at[pl.ds(sid * _EPT, _EPT)], dstv)
        pltpu.sync_copy(s_hbm.at[pl.ds(sid * _EPT, _EPT)], srcv)
        for half in range(2):
            lo = cid * (2 * _CHUNK) + half * _CHUNK
            zcopies = [
                pltpu.async_copy(
                    zbuf, csh.at[pl.ds(sid * _SLICE + z * _ZW, _ZW)], zsem)
                for z in range(_SLICE // _ZW)
            ]

            lo_v = jnp.full((16,), lo, jnp.int32)
            chunk_v = jnp.full((16,), _CHUNK, jnp.int32)
            zero_i = jnp.zeros((16,), jnp.int32)
            zero_f = jnp.zeros((16,), jnp.float32)
            one_f = jnp.ones((16,), jnp.float32)
            dump_v = jnp.full((16,), sid * 16 + _CW, jnp.int32) \
                + lax.iota(jnp.int32, 16)

            def outer(b, c):
                def inner(o, c2):
                    base = b * 128 + o * 16
                    d = dstv[pl.ds(base, 16)]
                    s = srcv[pl.ds(base, 16)]
                    rel = d - lo_v
                    m = (rel >= zero_i) & (rel < chunk_v)
                    idxv[b, pl.ds(o * 16, 16)] = jnp.where(m, rel * N + s,
                                                           dump_v)
                    valv[b, pl.ds(o * 16, 16)] = jnp.where(m, one_f, zero_f)
                    return c2
                return lax.fori_loop(0, 8, inner, c)
            lax.fori_loop(0, 32, outer, 0)
            for cp in zcopies:
                cp.wait()
            plsc.subcore_barrier()

            scopies = [
                pltpu.async_copy(valv.at[b], csh.at[idxv.at[b]], ssem,
                                 add=True)
                for b in range(32)
            ]
            for cp in scopies:
                cp.wait()
            plsc.subcore_barrier()

            row0 = lo + sid * (_CHUNK // _NS)
            pltpu.sync_copy(csh.at[pl.ds(sid * _SLICE, _SLICE)],
                            out_hbm.at[pl.ds(row0 * N, _SLICE)])


def _cbuild(src, dst):
    mesh = plsc.VectorSubcoreMesh(core_axis_name="c", subcore_axis_name="s")
    f = functools.partial(
        pl.kernel, mesh=mesh,
        out_type=jax.ShapeDtypeStruct((N * N,), jnp.float32),
        scratch_types=[
            pltpu.VMEM((_EPT,), jnp.int32),
            pltpu.VMEM((_EPT,), jnp.int32),
            pltpu.VMEM((32, 128), jnp.int32),
            pltpu.VMEM((32, 128), jnp.float32),
            pltpu.VMEM((_ZW,), jnp.float32),
            pltpu.VMEM_SHARED((_CW + 256,), jnp.float32),
            pltpu.SemaphoreType.DMA,
            pltpu.SemaphoreType.DMA,
        ],
    )(_cbuild_body)
    return f(src, dst).reshape(N, N)


def _embed_gather_body(tok_s, pos_s, tok_t, pos_t, semb, temb, ptab,
                       o_es, o_ps, o_et, o_pt, idxv, rowsv, sem):
    cid = lax.axis_index("c")
    sid = lax.axis_index("s")
    base = (sid * _NC + cid) * _RPT
    for ids, tab, out in ((tok_s, semb, o_es), (pos_s, ptab, o_ps),
                          (tok_t, temb, o_et), (pos_t, ptab, o_pt)):
        pltpu.sync_copy(ids.at[pl.ds(base, _RPT)], idxv)
        pltpu.async_copy(tab.at[idxv], rowsv, sem).wait()
        pltpu.sync_copy(rowsv, out.at[pl.ds(base, _RPT)])


def _embed_gather(src_tokens, src_pos, tgt_tokens, tgt_pos,
                  src_emb, tgt_emb, pos_table):
    mesh = plsc.VectorSubcoreMesh(core_axis_name="c", subcore_axis_name="s")
    f = functools.partial(
        pl.kernel, mesh=mesh,
        out_type=[jax.ShapeDtypeStruct((N, D), jnp.float32)] * 4,
        scratch_types=[
            pltpu.VMEM((_RPT,), jnp.int32),
            pltpu.VMEM((_RPT, D), jnp.float32),
            pltpu.SemaphoreType.DMA,
        ],
    )(_embed_gather_body)
    return f(src_tokens, src_pos, tgt_tokens, tgt_pos,
             src_emb, tgt_emb, pos_table)


def _embed_combine_body(es_ref, ps_ref, et_ref, pt_ref, xe_ref, xd_ref):
    xe_ref[...] = es_ref[...] * SQRT_D + ps_ref[...]
    xd_ref[...] = et_ref[...] * SQRT_D + pt_ref[...]


def _embed_combine(es, ps, et, pt):
    return pl.pallas_call(
        _embed_combine_body,
        out_shape=(jax.ShapeDtypeStruct((N, D), jnp.float32),
                   jax.ShapeDtypeStruct((N, D), jnp.float32)),
        interpret=_INTERP,
    )(es, ps, et, pt)


def _ln_mm_split_body(x_ref, w_ref, *out_refs):
    y = _dot16(_ln(x_ref[...]), w_ref[...])
    off = 0
    for r in out_refs:
        r[...] = y[:, off:off + r.shape[1]]
        off += r.shape[1]


def _ln_mm_split(x, w, n_out):
    kw = w.shape[1] // n_out
    return pl.pallas_call(
        _ln_mm_split_body,
        out_shape=tuple(jax.ShapeDtypeStruct((x.shape[0], kw), jnp.float32)
                        for _ in range(n_out)),
        interpret=_INTERP,
    )(x, w)


def _attn_o(q_ref, k_ref, v_ref, c_ref):
    c = c_ref[...]
    ohs = []
    for h in range(H):
        qh = q_ref[:, h * DK:(h + 1) * DK]
        kh = k_ref[:, h * DK:(h + 1) * DK]
        vh = v_ref[:, h * DK:(h + 1) * DK]
        s = jax.lax.dot_general(qh.astype(jnp.bfloat16),
                                kh.astype(jnp.bfloat16),
                                (((1,), (1,)), ((), ())),
                                preferred_element_type=jnp.float32)
        a = jnp.exp(jnp.clip(s * INV_SQRT_DK, -10.0, 10.0)) * c
        z = jnp.sum(a, axis=1, keepdims=True)
        wv = jnp.dot(a, vh, preferred_element_type=jnp.float32)
        ohs.append(wv / (z + 1e-6))
    return jnp.concatenate(ohs, axis=1)


def _attn_ffn_body(q_ref, k_ref, v_ref, c_ref, x_ref, wo_ref, w1_ref,
                   w2_ref, out_ref):
    o = _attn_o(q_ref, k_ref, v_ref, c_ref)
    x2 = x_ref[...] + _dot16(o, wo_ref[...])
    hh = jax.nn.relu(_dot16(_ln(x2), w1_ref[...]))
    out_ref[...] = x2 + _dot16(hh, w2_ref[...])


def _attn_ffn(q, k, v, c, x, wo, w1, w2, bd=512):
    return pl.pallas_call(
        _attn_ffn_body,
        grid=(N // bd,),
        in_specs=[
            pl.BlockSpec((bd, D), lambda i: (i, 0)),
            pl.BlockSpec((N, D), lambda i: (0, 0)),
            pl.BlockSpec((N, D), lambda i: (0, 0)),
            pl.BlockSpec((bd, N), lambda i: (i, 0)),
            pl.BlockSpec((bd, D), lambda i: (i, 0)),
            pl.BlockSpec((D, D), lambda i: (0, 0)),
            pl.BlockSpec((D, DFF), lambda i: (0, 0)),
            pl.BlockSpec((DFF, D), lambda i: (0, 0)),
        ],
        out_specs=pl.BlockSpec((bd, D), lambda i: (i, 0)),
        out_shape=jax.ShapeDtypeStruct((N, D), jnp.float32),
        interpret=_INTERP,
    )(q, k, v, c, x, wo, w1, w2)


def _attn_resq_body(q_ref, k_ref, v_ref, c_ref, x_ref, wo_ref, wq_ref,
                    x2_ref, qd_ref):
    o = _attn_o(q_ref, k_ref, v_ref, c_ref)
    x2 = x_ref[...] + _dot16(o, wo_ref[...])
    x2_ref[...] = x2
    qd_ref[...] = _dot16(_ln(x2), wq_ref[...])


def _attn_resq(q, k, v, c, x, wo, wq, bd=512):
    return pl.pallas_call(
        _attn_resq_body,
        grid=(N // bd,),
        in_specs=[
            pl.BlockSpec((bd, D), lambda i: (i, 0)),
            pl.BlockSpec((N, D), lambda i: (0, 0)),
            pl.BlockSpec((N, D), lambda i: (0, 0)),
            pl.BlockSpec((bd, N), lambda i: (i, 0)),
            pl.BlockSpec((bd, D), lambda i: (i, 0)),
            pl.BlockSpec((D, D), lambda i: (0, 0)),
            pl.BlockSpec((D, D), lambda i: (0, 0)),
        ],
        out_specs=(pl.BlockSpec((bd, D), lambda i: (i, 0)),
                   pl.BlockSpec((bd, D), lambda i: (i, 0))),
        out_shape=(jax.ShapeDtypeStruct((N, D), jnp.float32),
                   jax.ShapeDtypeStruct((N, D), jnp.float32)),
        interpret=_INTERP,
    )(q, k, v, c, x, wo, wq)


def _res_ffn_body(x_ref, o_ref, wo_ref, w1_ref, w2_ref, out_ref):
    x2 = x_ref[...] + _dot16(o_ref[...], wo_ref[...])
    hh = jax.nn.relu(_dot16(_ln(x2), w1_ref[...]))
    out_ref[...] = x2 + _dot16(hh, w2_ref[...])


def _res_ffn(x, o, wo, w1, w2):
    return pl.pallas_call(
        _res_ffn_body,
        out_shape=jax.ShapeDtypeStruct((N, D), jnp.float32),
        interpret=_INTERP,
    )(x, o, wo, w1, w2)


def _res_q_body(x_ref, o_ref, wo_ref, wq_ref, x2_ref, q_ref):
    x2 = x_ref[...] + _dot16(o_ref[...], wo_ref[...])
    x2_ref[...] = x2
    q_ref[...] = _dot16(_ln(x2), wq_ref[...])


def _res_q(x, o, wo, wq):
    return pl.pallas_call(
        _res_q_body,
        out_shape=(jax.ShapeDtypeStruct((N, D), jnp.float32),
                   jax.ShapeDtypeStruct((N, D), jnp.float32)),
        interpret=_INTERP,
    )(x, o, wo, wq)


def _gen_body(x_ref, w_ref, out_ref):
    logits = _dot16(x_ref[...], w_ref[...])
    m = jnp.max(logits, axis=1, keepdims=True)
    lse = m + jnp.log(jnp.sum(jnp.exp(logits - m), axis=1, keepdims=True))
    out_ref[...] = logits - lse


def _gen(x, w, br=256):
    return pl.pallas_call(
        _gen_body,
        grid=(N // br,),
        in_specs=[
            pl.BlockSpec((br, D), lambda i: (i, 0)),
            pl.BlockSpec((D, VOCAB), lambda i: (0, 0)),
        ],
        out_specs=pl.BlockSpec((br, VOCAB), lambda i: (i, 0)),
        out_shape=jax.ShapeDtypeStruct((N, VOCAB), jnp.float32),
        interpret=_INTERP,
    )(x, w)


def kernel(src_tokens, tgt_tokens, src_pos, tgt_pos, ee_src, ee_dst,
           dd_src, dd_dst, ed_src, ed_dst, src_emb, tgt_emb, pos_table,
           enc_Wqkv, enc_Wo, enc_W1, enc_W2, dec_Wqkv, dec_Wo1, dec_Wq,
           dec_Wkv, dec_Wo2, dec_W1, dec_W2, gen_W):
    es, ps, et, pt = _embed_gather(src_tokens, src_pos, tgt_tokens, tgt_pos,
                                   src_emb, tgt_emb, pos_table)
    x_enc, x_dec = _embed_combine(es, ps, et, pt)

    c_ee = _cbuild(ee_src, ee_dst)
    c_dd = _cbuild(dd_src, dd_dst)
    c_ed = _cbuild(ed_src, ed_dst)

    for i in range(2):
        q, k, v = _ln_mm_split(x_enc, enc_Wqkv[i], 3)
        x_enc = _attn_ffn(q, k, v, c_ee, x_enc, enc_Wo[i],
                          enc_W1[i], enc_W2[i])

    for i in range(2):
        q, k, v = _ln_mm_split(x_dec, dec_Wqkv[i], 3)
        x_dec, qd = _attn_resq(q, k, v, c_dd, x_dec, dec_Wo1[i], dec_Wq[i])
        kk, vv = _ln_mm_split(x_enc, dec_Wkv[i], 2)
        x_dec = _attn_ffn(qd, kk, vv, c_ed, x_dec, dec_Wo2[i],
                          dec_W1[i], dec_W2[i])

    return _gen(x_dec, gen_W)


# speedup vs baseline: 1.1099x; 1.1099x over previous
import functools

import jax
import jax.numpy as jnp
import numpy as np
from jax import lax
from jax.experimental import pallas as pl
from jax.experimental.pallas import tpu as pltpu
from jax.experimental.pallas import tpu_sc as plsc

N = 2048
D = 256
H = 8
DK = 32
DFF = 1024
VOCAB = 8192
E = 65536
SQRT_D = float(np.sqrt(D))
INV_SQRT_DK = float(1.0 / np.sqrt(DK))

_INTERP = False

_NC = 2
_NS = 16
_EPT = E // _NS
_CHUNK = 512
_CW = _CHUNK * N
_SLICE = _CW // _NS
_ZW = 8192
_RPT = N // (_NC * _NS)


def _dot16(a, b):
    return jax.lax.dot_general(
        a.astype(jnp.bfloat16), b.astype(jnp.bfloat16),
        (((1,), (0,)), ((), ())), preferred_element_type=jnp.float32)


def _ln(x):
    m = jnp.mean(x, axis=-1, keepdims=True)
    v = jnp.mean((x - m) ** 2, axis=-1, keepdims=True)
    return (x - m) * jax.lax.rsqrt(v + 1e-5)


def _cbuild_body(ees, eed, cee,
                 dstv, srcv, idxv, valv, zbuf, csh, zsem, ssem):
    cid = lax.axis_index("c")
    sid = lax.axis_index("s")
    zero16 = jnp.zeros((16,), jnp.float32)

    def zb(i, c):
        zbuf[pl.ds(i * 16, 16)] = zero16
        return c
    lax.fori_loop(0, _ZW // 16, zb, 0)

    for s_hbm, d_hbm, out_hbm in ((ees, eed, cee),):
        pltpu.sync_copy(d_hbm.at[pl.ds(sid * _EPT, _EPT)], dstv)
        pltpu.sync_copy(s_hbm.at[pl.ds(sid * _EPT, _EPT)], srcv)
        for half in range(2):
            lo = cid * (2 * _CHUNK) + half * _CHUNK
            zcopies = [
                pltpu.async_copy(
                    zbuf, csh.at[pl.ds(sid * _SLICE + z * _ZW, _ZW)], zsem)
                for z in range(_SLICE // _ZW)
            ]

            lo_v = jnp.full((16,), lo, jnp.int32)
            chunk_v = jnp.full((16,), _CHUNK, jnp.int32)
            zero_i = jnp.zeros((16,), jnp.int32)
            zero_f = jnp.zeros((16,), jnp.float32)
            one_f = jnp.ones((16,), jnp.float32)
            dump_v = jnp.full((16,), sid * 16 + _CW, jnp.int32) \
                + lax.iota(jnp.int32, 16)

            def outer(b, c):
                def inner(o, c2):
                    base = b * 128 + o * 16
                    d = dstv[pl.ds(base, 16)]
                    s = srcv[pl.ds(base, 16)]
                    rel = d - lo_v
                    m = (rel >= zero_i) & (rel < chunk_v)
                    idxv[b, pl.ds(o * 16, 16)] = jnp.where(m, rel * N + s,
                                                           dump_v)
                    valv[b, pl.ds(o * 16, 16)] = jnp.where(m, one_f, zero_f)
                    return c2
                return lax.fori_loop(0, 8, inner, c)
            lax.fori_loop(0, 32, outer, 0)
            for cp in zcopies:
                cp.wait()
            plsc.subcore_barrier()

            scopies = [
                pltpu.async_copy(valv.at[b], csh.at[idxv.at[b]], ssem,
                                 add=True)
                for b in range(32)
            ]
            for cp in scopies:
                cp.wait()
            plsc.subcore_barrier()

            row0 = lo + sid * (_CHUNK // _NS)
            pltpu.sync_copy(csh.at[pl.ds(sid * _SLICE, _SLICE)],
                            out_hbm.at[pl.ds(row0 * N, _SLICE)])


def _cbuild(src, dst):
    mesh = plsc.VectorSubcoreMesh(core_axis_name="c", subcore_axis_name="s")
    f = functools.partial(
        pl.kernel, mesh=mesh,
        out_type=jax.ShapeDtypeStruct((N * N,), jnp.float32),
        scratch_types=[
            pltpu.VMEM((_EPT,), jnp.int32),
            pltpu.VMEM((_EPT,), jnp.int32),
            pltpu.VMEM((32, 128), jnp.int32),
            pltpu.VMEM((32, 128), jnp.float32),
            pltpu.VMEM((_ZW,), jnp.float32),
            pltpu.VMEM_SHARED((_CW + 256,), jnp.float32),
            pltpu.SemaphoreType.DMA,
            pltpu.SemaphoreType.DMA,
        ],
    )(_cbuild_body)
    return f(src, dst).reshape(N, N)


def _embed_gather_body(tok_s, pos_s, tok_t, pos_t, semb, temb, ptab,
                       o_es, o_ps, o_et, o_pt, idxv, rowsv, sem):
    cid = lax.axis_index("c")
    sid = lax.axis_index("s")
    base = (sid * _NC + cid) * _RPT
    for ids, tab, out in ((tok_s, semb, o_es), (pos_s, ptab, o_ps),
                          (tok_t, temb, o_et), (pos_t, ptab, o_pt)):
        pltpu.sync_copy(ids.at[pl.ds(base, _RPT)], idxv)
        pltpu.async_copy(tab.at[idxv], rowsv, sem).wait()
        pltpu.sync_copy(rowsv, out.at[pl.ds(base, _RPT)])


def _embed_gather(src_tokens, src_pos, tgt_tokens, tgt_pos,
                  src_emb, tgt_emb, pos_table):
    mesh = plsc.VectorSubcoreMesh(core_axis_name="c", subcore_axis_name="s")
    f = functools.partial(
        pl.kernel, mesh=mesh,
        out_type=[jax.ShapeDtypeStruct((N, D), jnp.float32)] * 4,
        scratch_types=[
            pltpu.VMEM((_RPT,), jnp.int32),
            pltpu.VMEM((_RPT, D), jnp.float32),
            pltpu.SemaphoreType.DMA,
        ],
    )(_embed_gather_body)
    return f(src_tokens, src_pos, tgt_tokens, tgt_pos,
             src_emb, tgt_emb, pos_table)


def _embed_combine_body(es_ref, ps_ref, et_ref, pt_ref, xe_ref, xd_ref):
    xe_ref[...] = es_ref[...] * SQRT_D + ps_ref[...]
    xd_ref[...] = et_ref[...] * SQRT_D + pt_ref[...]


def _embed_combine(es, ps, et, pt):
    return pl.pallas_call(
        _embed_combine_body,
        out_shape=(jax.ShapeDtypeStruct((N, D), jnp.float32),
                   jax.ShapeDtypeStruct((N, D), jnp.float32)),
        interpret=_INTERP,
    )(es, ps, et, pt)


def _ln_mm_split_body(x_ref, w_ref, *out_refs):
    y = _dot16(_ln(x_ref[...]), w_ref[...])
    off = 0
    for r in out_refs:
        r[...] = y[:, off:off + r.shape[1]]
        off += r.shape[1]


def _ln_mm_split(x, w, n_out):
    kw = w.shape[1] // n_out
    return pl.pallas_call(
        _ln_mm_split_body,
        out_shape=tuple(jax.ShapeDtypeStruct((x.shape[0], kw), jnp.float32)
                        for _ in range(n_out)),
        interpret=_INTERP,
    )(x, w)


def _attn_body(q_ref, k_ref, v_ref, c_ref, o_ref):
    c = c_ref[...]
    for h in range(H):
        qh = q_ref[:, h * DK:(h + 1) * DK]
        kh = k_ref[:, h * DK:(h + 1) * DK]
        vh = v_ref[:, h * DK:(h + 1) * DK]
        s = jax.lax.dot_general(qh.astype(jnp.bfloat16),
                                kh.astype(jnp.bfloat16),
                                (((1,), (1,)), ((), ())),
                                preferred_element_type=jnp.float32)
        a = jnp.exp(jnp.clip(s * INV_SQRT_DK, -10.0, 10.0)) * c
        z = jnp.sum(a, axis=1, keepdims=True)
        wv = jnp.dot(a, vh, preferred_element_type=jnp.float32)
        o_ref[:, h * DK:(h + 1) * DK] = wv / (z + 1e-6)


def _attn(q, k, v, c, bd=512):
    return pl.pallas_call(
        _attn_body,
        grid=(N // bd,),
        in_specs=[
            pl.BlockSpec((bd, D), lambda i: (i, 0)),
            pl.BlockSpec((N, D), lambda i: (0, 0)),
            pl.BlockSpec((N, D), lambda i: (0, 0)),
            pl.BlockSpec((bd, N), lambda i: (i, 0)),
        ],
        out_specs=pl.BlockSpec((bd, D), lambda i: (i, 0)),
        out_shape=jax.ShapeDtypeStruct((N, D), jnp.float32),
        interpret=_INTERP,
    )(q, k, v, c)


def _res_ffn_body(x_ref, o_ref, wo_ref, w1_ref, w2_ref, out_ref):
    x2 = x_ref[...] + _dot16(o_ref[...], wo_ref[...])
    hh = jax.nn.relu(_dot16(_ln(x2), w1_ref[...]))
    out_ref[...] = x2 + _dot16(hh, w2_ref[...])


def _res_ffn(x, o, wo, w1, w2):
    return pl.pallas_call(
        _res_ffn_body,
        out_shape=jax.ShapeDtypeStruct((N, D), jnp.float32),
        interpret=_INTERP,
    )(x, o, wo, w1, w2)


def _res_q_body(x_ref, o_ref, wo_ref, wq_ref, x2_ref, q_ref):
    x2 = x_ref[...] + _dot16(o_ref[...], wo_ref[...])
    x2_ref[...] = x2
    q_ref[...] = _dot16(_ln(x2), wq_ref[...])


def _res_q(x, o, wo, wq):
    return pl.pallas_call(
        _res_q_body,
        out_shape=(jax.ShapeDtypeStruct((N, D), jnp.float32),
                   jax.ShapeDtypeStruct((N, D), jnp.float32)),
        interpret=_INTERP,
    )(x, o, wo, wq)


def _gen_body(x_ref, w_ref, out_ref):
    logits = _dot16(x_ref[...], w_ref[...])
    m = jnp.max(logits, axis=1, keepdims=True)
    lse = m + jnp.log(jnp.sum(jnp.exp(logits - m), axis=1, keepdims=True))
    out_ref[...] = logits - lse


def _gen(x, w, br=512):
    return pl.pallas_call(
        _gen_body,
        grid=(N // br,),
        in_specs=[
            pl.BlockSpec((br, D), lambda i: (i, 0)),
            pl.BlockSpec((D, VOCAB), lambda i: (0, 0)),
        ],
        out_specs=pl.BlockSpec((br, VOCAB), lambda i: (i, 0)),
        out_shape=jax.ShapeDtypeStruct((N, VOCAB), jnp.float32),
        interpret=_INTERP,
    )(x, w)


def kernel(src_tokens, tgt_tokens, src_pos, tgt_pos, ee_src, ee_dst,
           dd_src, dd_dst, ed_src, ed_dst, src_emb, tgt_emb, pos_table,
           enc_Wqkv, enc_Wo, enc_W1, enc_W2, dec_Wqkv, dec_Wo1, dec_Wq,
           dec_Wkv, dec_Wo2, dec_W1, dec_W2, gen_W):
    es, ps, et, pt = _embed_gather(src_tokens, src_pos, tgt_tokens, tgt_pos,
                                   src_emb, tgt_emb, pos_table)
    x_enc, x_dec = _embed_combine(es, ps, et, pt)

    c_ee = _cbuild(ee_src, ee_dst)
    c_dd = _cbuild(dd_src, dd_dst)
    c_ed = _cbuild(ed_src, ed_dst)

    for i in range(2):
        q, k, v = _ln_mm_split(x_enc, enc_Wqkv[i], 3)
        o = _attn(q, k, v, c_ee)
        x_enc = _res_ffn(x_enc, o, enc_Wo[i], enc_W1[i], enc_W2[i])

    for i in range(2):
        q, k, v = _ln_mm_split(x_dec, dec_Wqkv[i], 3)
        o = _attn(q, k, v, c_dd)
        x_dec, qd = _res_q(x_dec, o, dec_Wo1[i], dec_Wq[i])
        kk, vv = _ln_mm_split(x_enc, dec_Wkv[i], 2)
        o = _attn(qd, kk, vv, c_ed)
        x_dec = _res_ffn(x_dec, o, dec_Wo2[i], dec_W1[i], dec_W2[i])

    return _gen(x_dec, gen_W)
